# baseline (speedup 1.0000x reference)
import jax
import jax.numpy as jnp
from jax import lax
from jax.experimental import pallas as pl
from jax.experimental.pallas import tpu as pltpu
from jax.experimental.pallas import tpu_sc as plsc

EMBED = 64
SEQ = 200
BATCH = 4096
NUM_CORES = 2
NUM_SUBCORES = 16
NW = NUM_CORES * NUM_SUBCORES
BT = BATCH // NW
NBUF = 2


def _gather_body(table_hbm, sent_hbm, out_hbm, sent_v, rows_v, rowsT_v,
                 sem_g, sem_out):
    w = lax.axis_index("s") * NUM_CORES + lax.axis_index("c")
    b0 = pl.multiple_of(w * BT, BT)

    pltpu.sync_copy(sent_hbm.at[:, pl.ds(b0, BT)], sent_v)

    bvecs = [lax.iota(jnp.int32, 16) + bb for bb in range(0, BT, 16)]

    def fire_gather(s, p):
        return pltpu.async_copy(
            table_hbm.at[sent_v.at[s]], rows_v.at[p], sem_g)

    def transpose(p):
        src = rows_v.at[p]
        dst = rowsT_v.at[p]

        def dt_body(dt, carry):
            for d8 in range(8):
                d = dt * 8 + d8
                dvec = jnp.full((16,), d, jnp.int32)
                for j in range(BT // 16):
                    vec = plsc.load_gather(src, [bvecs[j], dvec])
                    dst[dt, d8, pl.ds(j * 16, 16)] = vec
            return carry

        lax.fori_loop(0, 8, dt_body, 0)

    fire_gather(0, 0)

    def body(s, carry):
        p = lax.rem(s, NBUF)

        @pl.when(s + 1 < SEQ)
        def _():
            fire_gather(s + 1, 1 - p)

        pltpu.make_async_copy(
            table_hbm.at[sent_v.at[0]], rows_v.at[p], sem_g).wait()

        @pl.when(s >= NBUF)
        def _():
            pltpu.make_async_copy(
                rowsT_v.at[p], out_hbm.at[0, :, 0], sem_out).wait()

        transpose(p)

        pltpu.async_copy(rowsT_v.at[p], out_hbm.at[s, :, w], sem_out)
        return carry

    lax.fori_loop(0, SEQ, body, 0)

    for _ in range(NBUF):
        pltpu.make_async_copy(
            rowsT_v.at[0], out_hbm.at[0, :, 0], sem_out).wait()


def kernel(sentence, word_embed_weight):
    batch, seq = sentence.shape
    vocab, embed = word_embed_weight.shape
    sent_t = sentence.T
    mesh = plsc.VectorSubcoreMesh(core_axis_name="c", subcore_axis_name="s")
    run = pl.kernel(
        _gather_body,
        out_type=jax.ShapeDtypeStruct((seq, 8, NW, 8, BT), jnp.float32),
        mesh=mesh,
        scratch_types=[
            pltpu.VMEM((seq, BT), jnp.int32),
            pltpu.VMEM((NBUF, BT, embed), jnp.float32),
            pltpu.VMEM((NBUF, 8, 8, BT), jnp.float32),
            pltpu.SemaphoreType.DMA,
            pltpu.SemaphoreType.DMA,
        ],
        compiler_params=pltpu.CompilerParams(
            use_tc_tiling_on_sc=False, needs_layout_passes=False),
    )
    out5 = run(word_embed_weight, sent_t)
    out = out5.transpose(2, 4, 0, 1, 3).reshape(batch, seq, embed)
    return out


# speedup vs baseline: 1.0306x; 1.0306x over previous
import jax
import jax.numpy as jnp
from jax import lax
from jax.experimental import pallas as pl
from jax.experimental.pallas import tpu as pltpu
from jax.experimental.pallas import tpu_sc as plsc

EMBED = 64
SEQ = 200
BATCH = 4096
NUM_CORES = 2
NUM_SUBCORES = 16
NW = NUM_CORES * NUM_SUBCORES
BT = BATCH // NW
NBUF = 2


def _gather_body(table_hbm, sent_hbm, out_hbm, sent_v, rows_v, rowsT_v,
                 sem_g, sem_out):
    w = lax.axis_index("s") * NUM_CORES + lax.axis_index("c")
    b0 = pl.multiple_of(w * BT, BT)

    pltpu.sync_copy(sent_hbm.at[:, pl.ds(b0, BT)], sent_v)

    def dbl_body(s, carry):
        for bb in range(0, BT, 16):
            v = sent_v[s, pl.ds(bb, 16)]
            sent_v[s, pl.ds(bb, 16)] = v + v
        return carry

    lax.fori_loop(0, SEQ, dbl_body, 0)

    bvecs = [lax.iota(jnp.int32, 16) + bb for bb in range(0, BT, 16)]

    def fire_gather(s, p):
        return pltpu.async_copy(
            table_hbm.at[sent_v.at[s]], rows_v.at[p], sem_g)

    def transpose(p):
        src = rows_v.at[p]
        dst = rowsT_v.at[p]

        def dt_body(dt, carry):
            for d8 in range(8):
                d = dt * 8 + d8
                dvec = jnp.full((16,), d, jnp.int32)
                for j in range(BT // 16):
                    vec = plsc.load_gather(src, [bvecs[j], dvec])
                    dst[dt, d8, pl.ds(j * 16, 16)] = vec
            return carry

        lax.fori_loop(0, 8, dt_body, 0)

    fire_gather(0, 0)

    def body(s, carry):
        p = lax.rem(s, NBUF)

        @pl.when(s + 1 < SEQ)
        def _():
            fire_gather(s + 1, 1 - p)

        pltpu.make_async_copy(
            table_hbm.at[sent_v.at[0]], rows_v.at[p], sem_g).wait()

        @pl.when(s >= NBUF)
        def _():
            pltpu.make_async_copy(
                rowsT_v.at[p], out_hbm.at[0, :, 0], sem_out).wait()

        transpose(p)

        pltpu.async_copy(rowsT_v.at[p], out_hbm.at[s, :, w], sem_out)
        return carry

    lax.fori_loop(0, SEQ, body, 0)

    for _ in range(NBUF):
        pltpu.make_async_copy(
            rowsT_v.at[0], out_hbm.at[0, :, 0], sem_out).wait()


def kernel(sentence, word_embed_weight):
    batch, seq = sentence.shape
    vocab, embed = word_embed_weight.shape
    sent_t = sentence.T
    tab_pad = jnp.pad(word_embed_weight, ((0, 0), (0, 128 - embed)))
    tab2 = tab_pad.reshape(2 * vocab, embed)
    mesh = plsc.VectorSubcoreMesh(core_axis_name="c", subcore_axis_name="s")
    run = pl.kernel(
        _gather_body,
        out_type=jax.ShapeDtypeStruct((seq, 8, NW, 8, BT), jnp.float32),
        mesh=mesh,
        scratch_types=[
            pltpu.VMEM((seq, BT), jnp.int32),
            pltpu.VMEM((NBUF, BT, embed), jnp.float32),
            pltpu.VMEM((NBUF, 8, 8, BT), jnp.float32),
            pltpu.SemaphoreType.DMA,
            pltpu.SemaphoreType.DMA,
        ],
        compiler_params=pltpu.CompilerParams(
            use_tc_tiling_on_sc=False, needs_layout_passes=False),
    )
    out5 = run(tab2, sent_t)
    out = out5.transpose(2, 4, 0, 1, 3).reshape(batch, seq, embed)
    return out


# speedup vs baseline: 1.1941x; 1.1586x over previous
import jax
import jax.numpy as jnp
from jax import lax
from jax.experimental import pallas as pl
from jax.experimental.pallas import tpu as pltpu
from jax.experimental.pallas import tpu_sc as plsc

EMBED = 64
SEQ = 200
BATCH = 4096
NUM_CORES = 2
NUM_SUBCORES = 16
NW = NUM_CORES * NUM_SUBCORES
BT = BATCH // NW
SG = 2
STEPS = SEQ // SG
NBUF = 2


def _gather_body(table_hbm, sent_hbm, out_hbm, sent_v, rows_v, rowsT_v,
                 sem_g, sem_out):
    w = lax.axis_index("s") * NUM_CORES + lax.axis_index("c")
    b0 = pl.multiple_of(w * BT, BT)

    pltpu.sync_copy(sent_hbm.at[:, pl.ds(b0, BT)], sent_v)

    def dbl_body(s, carry):
        for bb in range(0, BT, 16):
            v = sent_v[s, pl.ds(bb, 16)]
            sent_v[s, pl.ds(bb, 16)] = v + v
        return carry

    lax.fori_loop(0, SEQ, dbl_body, 0)

    iota = lax.iota(jnp.int32, 16)
    dtv = [(iota + 16 * j) >> 3 for j in range(EMBED // 16)]
    d8v = [(iota + 16 * j) & 7 for j in range(EMBED // 16)]

    def fire_gathers(i, p):
        for k in range(SG):
            pltpu.async_copy(
                table_hbm.at[sent_v.at[i * SG + k]],
                rows_v.at[p].at[k], sem_g)

    def drain_gathers(p):
        for k in range(SG):
            pltpu.make_async_copy(
                table_hbm.at[sent_v.at[0]], rows_v.at[p].at[k], sem_g).wait()

    def transpose(p):
        for k in range(SG):
            src = rows_v.at[p].at[k]
            dst = rowsT_v.at[p].at[k]

            def b_body(b, carry):
                bvec = jnp.full((16,), b, jnp.int32)
                for j in range(EMBED // 16):
                    vec = src[b, pl.ds(16 * j, 16)]
                    plsc.store_scatter(dst, [dtv[j], d8v[j], bvec], vec)
                return carry

            lax.fori_loop(0, BT, b_body, 0, unroll=8)

    fire_gathers(0, 0)

    def body(i, carry):
        p = lax.rem(i, NBUF)

        @pl.when(i + 1 < STEPS)
        def _():
            fire_gathers(i + 1, 1 - p)

        drain_gathers(p)

        @pl.when(i >= NBUF)
        def _():
            pltpu.make_async_copy(
                rowsT_v.at[p], out_hbm.at[pl.ds(0, SG), :, 0], sem_out).wait()

        transpose(p)

        pltpu.async_copy(
            rowsT_v.at[p], out_hbm.at[pl.ds(i * SG, SG), :, w], sem_out)
        return carry

    lax.fori_loop(0, STEPS, body, 0)

    for _ in range(NBUF):
        pltpu.make_async_copy(
            rowsT_v.at[0], out_hbm.at[pl.ds(0, SG), :, 0], sem_out).wait()


def kernel(sentence, word_embed_weight):
    batch, seq = sentence.shape
    vocab, embed = word_embed_weight.shape
    sent_t = sentence.T
    tab_pad = jnp.pad(word_embed_weight, ((0, 0), (0, 128 - embed)))
    tab2 = tab_pad.reshape(2 * vocab, embed)
    mesh = plsc.VectorSubcoreMesh(core_axis_name="c", subcore_axis_name="s")
    run = pl.kernel(
        _gather_body,
        out_type=jax.ShapeDtypeStruct((seq, 8, NW, 8, BT), jnp.float32),
        mesh=mesh,
        scratch_types=[
            pltpu.VMEM((seq, BT), jnp.int32),
            pltpu.VMEM((NBUF, SG, BT, embed), jnp.float32),
            pltpu.VMEM((NBUF, SG, 8, 8, BT), jnp.float32),
            pltpu.SemaphoreType.DMA,
            pltpu.SemaphoreType.DMA,
        ],
        compiler_params=pltpu.CompilerParams(
            use_tc_tiling_on_sc=False, needs_layout_passes=False),
    )
    out5 = run(tab2, sent_t)
    out = out5.transpose(2, 4, 0, 1, 3).reshape(batch, seq, embed)
    return out


# speedup vs baseline: 2.2118x; 1.8522x over previous
import jax
import jax.numpy as jnp
from jax import lax
from jax.experimental import pallas as pl
from jax.experimental.pallas import tpu as pltpu
from jax.experimental.pallas import tpu_sc as plsc

EMBED = 64
SEQ = 200
BATCH = 4096
NUM_CORES = 2
NUM_SUBCORES = 16
NW = NUM_CORES * NUM_SUBCORES
BT = BATCH // NW
SG = 1
STEPS = SEQ // SG
NBUF = 2
GBUF = 3


def _gather_body(table_hbm, sent_hbm, out_hbm, sent_v, rows_v, rowsT_v,
                 sem_g, sem_out):
    w = lax.axis_index("s") * NUM_CORES + lax.axis_index("c")
    b0 = pl.multiple_of(w * BT, BT)

    pltpu.sync_copy(sent_hbm.at[:, pl.ds(b0, BT)], sent_v)

    def dbl_body(s, carry):
        for bb in range(0, BT, 16):
            v = sent_v[s, pl.ds(bb, 16)]
            sent_v[s, pl.ds(bb, 16)] = v + v
        return carry

    lax.fori_loop(0, SEQ, dbl_body, 0)

    iota = lax.iota(jnp.int32, 16)
    bv = [(iota + bb) * EMBED for bb in range(0, BT, 16)]

    def fire_gathers(i, p):
        for k in range(SG):
            pltpu.async_copy(
                table_hbm.at[sent_v.at[i * SG + k]],
                rows_v.at[p].at[k].at[:, pl.ds(0, EMBED)], sem_g)

    def drain_gathers(p):
        for k in range(SG):
            pltpu.make_async_copy(
                table_hbm.at[sent_v.at[0]],
                rows_v.at[p].at[k].at[:, pl.ds(0, EMBED)], sem_g).wait()

    zero = iota * 0

    def transpose(p):
        for k in range(SG):
            src = rows_v.at[p].at[k]
            dst = rowsT_v.at[p].at[k]
            for d in range(EMBED):
                for jb in range(BT // 16):
                    vec = plsc.load_gather(src, [iota + jb * 16, zero + d])
                    dst[d // 8, d % 8, pl.ds(jb * 16, 16)] = vec

    fire_gathers(0, 0)
    fire_gathers(1, 1)

    def body(i, carry):
        p = lax.rem(i, GBUF)
        q = lax.rem(i, NBUF)

        @pl.when(i + 2 < STEPS)
        def _():
            fire_gathers(i + 2, lax.rem(i + 2, GBUF))

        drain_gathers(p)

        @pl.when(i >= NBUF)
        def _():
            pltpu.make_async_copy(
                rowsT_v.at[p], out_hbm.at[pl.ds(0, SG), :, 0], sem_out).wait()

        transpose(p)

        pltpu.async_copy(
            rowsT_v.at[p].at[:, :, :, pl.ds(0, BT)],
            out_hbm.at[pl.ds(i * SG, SG), :, w], sem_out)
        return carry

    lax.fori_loop(0, STEPS, body, 0)

    for _ in range(NBUF):
        pltpu.make_async_copy(
            rowsT_v.at[0], out_hbm.at[pl.ds(0, SG), :, 0], sem_out).wait()


def kernel(sentence, word_embed_weight):
    batch, seq = sentence.shape
    vocab, embed = word_embed_weight.shape
    sent_t = sentence.T
    tab_pad = jnp.pad(word_embed_weight, ((0, 0), (0, 128 - embed)))
    tab2 = tab_pad.reshape(2 * vocab, embed)
    mesh = plsc.VectorSubcoreMesh(core_axis_name="c", subcore_axis_name="s")
    run = pl.kernel(
        _gather_body,
        out_type=jax.ShapeDtypeStruct((seq, 8, NW, 8, BT), jnp.float32),
        mesh=mesh,
        scratch_types=[
            pltpu.VMEM((seq, BT), jnp.int32),
            pltpu.VMEM((NBUF, SG, BT, embed + 1), jnp.float32),
            pltpu.VMEM((NBUF, SG, 8, 8, BT), jnp.float32),
            pltpu.SemaphoreType.DMA,
            pltpu.SemaphoreType.DMA,
        ],
        compiler_params=pltpu.CompilerParams(
            use_tc_tiling_on_sc=False, needs_layout_passes=False),
    )
    out5 = run(tab2, sent_t)
    out = out5.transpose(2, 4, 0, 1, 3).reshape(batch, seq, embed)
    return out


# speedup vs baseline: 2.2173x; 1.0025x over previous
import jax
import jax.numpy as jnp
from jax import lax
from jax.experimental import pallas as pl
from jax.experimental.pallas import tpu as pltpu
from jax.experimental.pallas import tpu_sc as plsc

EMBED = 64
SEQ = 200
BATCH = 4096
NUM_CORES = 2
NUM_SUBCORES = 16
NW = NUM_CORES * NUM_SUBCORES
BT = BATCH // NW
SG = 1
STEPS = SEQ // SG
NBUF = 2
GBUF = 3


def _gather_body(table_hbm, sent_hbm, out_hbm, sent_v, rows_v, rowsT_v,
                 sem_g, sem_out):
    w = lax.axis_index("s") * NUM_CORES + lax.axis_index("c")
    b0 = pl.multiple_of(w * BT, BT)

    pltpu.sync_copy(sent_hbm.at[:, pl.ds(b0, BT)], sent_v)

    def dbl_body(s, carry):
        for bb in range(0, BT, 16):
            v = sent_v[s, pl.ds(bb, 16)]
            sent_v[s, pl.ds(bb, 16)] = v + v
        return carry

    lax.fori_loop(0, SEQ, dbl_body, 0)

    iota = lax.iota(jnp.int32, 16)

    def fire_gathers(i, p):
        for k in range(SG):
            pltpu.async_copy(
                table_hbm.at[sent_v.at[i * SG + k]],
                rows_v.at[p].at[k].at[:, pl.ds(0, EMBED)], sem_g)

    def drain_gathers(p):
        for k in range(SG):
            pltpu.make_async_copy(
                table_hbm.at[sent_v.at[0]],
                rows_v.at[p].at[k].at[:, pl.ds(0, EMBED)], sem_g).wait()

    zero = iota * 0

    def transpose(p):
        for k in range(SG):
            src = rows_v.at[p].at[k]
            dst = rowsT_v.at[p].at[k]
            for d in range(EMBED):
                for jb in range(BT // 16):
                    vec = plsc.load_gather(src, [iota + jb * 16, zero + d])
                    dst[d // 8, d % 8, pl.ds(jb * 16, 16)] = vec

    fire_gathers(0, 0)
    fire_gathers(1, 1)

    def body(i, carry):
        p = lax.rem(i, GBUF)
        q = lax.rem(i, NBUF)

        @pl.when(i + 2 < STEPS)
        def _():
            fire_gathers(i + 2, lax.rem(i + 2, GBUF))

        drain_gathers(p)

        @pl.when(i >= NBUF)
        def _():
            pltpu.make_async_copy(
                rowsT_v.at[p], out_hbm.at[pl.ds(0, SG), :, 0], sem_out).wait()

        transpose(p)

        pltpu.async_copy(
            rowsT_v.at[p].at[:, :, :, pl.ds(0, BT)],
            out_hbm.at[pl.ds(i * SG, SG), :, w], sem_out)
        return carry

    lax.fori_loop(0, STEPS, body, 0)

    for _ in range(NBUF):
        pltpu.make_async_copy(
            rowsT_v.at[0], out_hbm.at[pl.ds(0, SG), :, 0], sem_out).wait()


def kernel(sentence, word_embed_weight):
    batch, seq = sentence.shape
    vocab, embed = word_embed_weight.shape
    sent_t = sentence.T
    tab_pad = jnp.pad(word_embed_weight, ((0, 0), (0, 128 - embed)))
    tab2 = tab_pad.reshape(2 * vocab, embed)
    mesh = plsc.VectorSubcoreMesh(core_axis_name="c", subcore_axis_name="s")
    run = pl.kernel(
        _gather_body,
        out_type=jax.ShapeDtypeStruct((seq, 8, NW, 8, BT), jnp.float32),
        mesh=mesh,
        scratch_types=[
            pltpu.VMEM((seq, BT), jnp.int32),
            pltpu.VMEM((NBUF, SG, BT, embed + 1), jnp.float32),
            pltpu.VMEM((NBUF, SG, 8, 8, BT), jnp.float32),
            pltpu.SemaphoreType.DMA,
            pltpu.SemaphoreType.DMA,
        ],
        compiler_params=pltpu.CompilerParams(
            use_tc_tiling_on_sc=False, needs_layout_passes=False),
    )
    out5 = run(tab2, sent_t)
    out = out5.transpose(2, 4, 0, 1, 3).reshape(batch, seq, embed)
    return out
